# Initial kernel scaffold; baseline (speedup 1.0000x reference)
import jax
import jax.numpy as jnp
from jax.experimental import pallas as pl


def kernel(x, edge_src, edge_dst, edge_w, weight, bias):
    raise NotImplementedError("write your pallas kernel here")


# speedup vs baseline: 1.8845x; 1.8845x over previous
import functools

import jax
import jax.numpy as jnp
from jax import lax
from jax.experimental import pallas as pl
from jax.experimental.pallas import tpu as pltpu
from jax.experimental.pallas import tpu_sc as plsc

B = 8
N = 5000
NP = 5120
E = 160000
EP = 163840
F = 128
K = 3
C = 4
CH = 128
NT = 16
RPT = NP // NT
RB = 64
EPT = EP // NT
NCH = EPT // CH
GPC = B // 2


def _sc_spmm(xflat, srcflat, dstflat, wflat):
    mesh = plsc.VectorSubcoreMesh(core_axis_name="c", subcore_axis_name="s")

    @functools.partial(
        pl.kernel,
        mesh=mesh,
        out_type=jax.ShapeDtypeStruct((B * K * NP, F), jnp.float32),
        scratch_types=[
            pltpu.VMEM((CH,), jnp.int32),
            pltpu.VMEM((CH,), jnp.int32),
            pltpu.VMEM((CH,), jnp.float32),
            pltpu.VMEM((CH, F), jnp.float32),
            pltpu.VMEM((RB, F), jnp.float32),
            pltpu.VMEM((RB, F), jnp.float32),
            pltpu.VMEM_SHARED((NP, F), jnp.float32),
            pltpu.SemaphoreType.DMA,
        ],
    )
    def k(x_hbm, src_hbm, dst_hbm, w_hbm, out_hbm,
          src_v, dst_v, w_v, rows_v, fin_v, prev_v, acc, sem):
        cid = lax.axis_index("c")
        sid = lax.axis_index("s")
        stripe = sid * RPT

        def zero_fin(r, carry):
            for f in range(F // 16):
                fin_v[r, pl.ds(f * 16, 16)] = jnp.zeros((16,), jnp.float32)
            return carry

        lax.fori_loop(0, RB, zero_fin, 0)

        def zero_acc(rb, carry):
            pltpu.sync_copy(fin_v, acc.at[pl.ds(stripe + rb * RB, RB)])
            return carry
        lax.fori_loop(0, RPT // RB, zero_acc, 0)
        plsc.subcore_barrier()

        def edge_pass(src_tab, src_row0):
            def chunk(j, carry):
                off = carry + j * CH
                pltpu.sync_copy(src_hbm.at[pl.ds(off, CH)], src_v)
                pltpu.sync_copy(dst_hbm.at[pl.ds(off, CH)], dst_v)
                pltpu.sync_copy(w_hbm.at[pl.ds(off, CH)], w_v)
                for i in range(CH // 16):
                    sl = pl.ds(i * 16, 16)
                    src_v[sl] = src_v[sl] + src_row0
                pltpu.async_copy(src_tab.at[src_v], rows_v, sem).wait()

                def scale(i, c2):
                    wvec = w_v[pl.ds(i * 16, 16)]
                    for j in range(16):
                        wv = wvec[j]
                        e = i * 16 + j
                        for f in range(F // 16):
                            sl = pl.ds(f * 16, 16)
                            rows_v[e, sl] = rows_v[e, sl] * wv
                    return c2
                lax.fori_loop(0, CH // 16, scale, 0)
                pltpu.sync_copy(rows_v, acc.at[dst_v], add=True)
                return carry
            return chunk

        def graph_body(g, carry):
            b = cid * GPC + g
            ebase = b * EP + sid * EPT

            for hop in range(K):
                if hop == 0:
                    src_tab, src_row0 = x_hbm, b * NP
                else:
                    src_tab, src_row0 = out_hbm, (b * K + hop - 1) * NP
                lax.fori_loop(0, NCH, edge_pass(src_tab, src_row0), ebase)
                plsc.subcore_barrier()

                out_row0 = (b * K + hop) * NP + stripe
                if hop == 0:
                    def fin0(rb, c2):
                        r0 = rb * RB
                        pltpu.sync_copy(acc.at[pl.ds(stripe + r0, RB)], fin_v)
                        pltpu.sync_copy(fin_v, out_hbm.at[pl.ds(out_row0 + r0, RB)])
                        lax.fori_loop(0, RB, zero_fin, 0)
                        pltpu.sync_copy(fin_v, acc.at[pl.ds(stripe + r0, RB)])
                        return c2
                    lax.fori_loop(0, RPT // RB, fin0, 0)
                else:
                    if hop == 1:
                        prev_tab, prev_row0 = x_hbm, b * NP + stripe
                    else:
                        prev_tab, prev_row0 = out_hbm, (b * K + hop - 2) * NP + stripe

                    def fin1(rb, c2):
                        r0 = rb * RB
                        pltpu.sync_copy(acc.at[pl.ds(stripe + r0, RB)], fin_v)
                        pltpu.sync_copy(prev_tab.at[pl.ds(prev_row0 + r0, RB)], prev_v)

                        def finrow(r, c3):
                            for f in range(F // 16):
                                sl = pl.ds(f * 16, 16)
                                prev_v[r, sl] = 2.0 * fin_v[r, sl] - prev_v[r, sl]
                                fin_v[r, sl] = jnp.zeros((16,), jnp.float32)
                            return c3
                        lax.fori_loop(0, RB, finrow, 0)
                        pltpu.sync_copy(prev_v, out_hbm.at[pl.ds(out_row0 + r0, RB)])
                        pltpu.sync_copy(fin_v, acc.at[pl.ds(stripe + r0, RB)])
                        return c2
                    lax.fori_loop(0, RPT // RB, fin1, 0)
                plsc.subcore_barrier()
            return carry

        lax.fori_loop(0, GPC, graph_body, 0)

    return k(xflat, srcflat, dstflat, wflat)


def _combine(x, hops, weight, bias):
    R = 1000

    def body(x_ref, h_ref, w_ref, b_ref, o_ref):
        xb = x_ref[0]
        for c in range(C):
            acc = xb * w_ref[c, 0, :][None, :]
            for kk in range(1, K + 1):
                acc = acc + h_ref[0, kk - 1] * w_ref[c, kk, :][None, :]
            o_ref[0, :, c * F:(c + 1) * F] = jnp.maximum(
                acc + b_ref[0, 0, c * F:(c + 1) * F][None, :], 0.0)

    return pl.pallas_call(
        body,
        grid=(B, N // R),
        in_specs=[
            pl.BlockSpec((1, R, F), lambda b, n: (b, n, 0)),
            pl.BlockSpec((1, K, R, F), lambda b, n: (b, 0, n, 0)),
            pl.BlockSpec((C, K + 1, F), lambda b, n: (0, 0, 0)),
            pl.BlockSpec((1, 1, C * F), lambda b, n: (0, 0, 0)),
        ],
        out_specs=pl.BlockSpec((1, R, C * F), lambda b, n: (b, n, 0)),
        out_shape=jax.ShapeDtypeStruct((B, N, C * F), jnp.float32),
    )(x, hops, weight, bias)


def kernel(x, edge_src, edge_dst, edge_w, weight, bias):
    x = x.astype(jnp.float32)
    xflat = jnp.pad(x, ((0, 0), (0, NP - N), (0, 0))).reshape(B * NP, F)
    srcflat = jnp.pad(edge_src.astype(jnp.int32), ((0, 0), (0, EP - E))).reshape(-1)
    dstflat = jnp.pad(edge_dst.astype(jnp.int32), ((0, 0), (0, EP - E))).reshape(-1)
    wflat = jnp.pad(edge_w, ((0, 0), (0, EP - E))).reshape(-1)
    hops = _sc_spmm(xflat, srcflat, dstflat, wflat).reshape(B, K, NP, F)
    return _combine(x, hops, weight, bias)


# speedup vs baseline: 2.9244x; 1.5519x over previous
import functools

import jax
import jax.numpy as jnp
from jax import lax
from jax.experimental import pallas as pl
from jax.experimental.pallas import tpu as pltpu
from jax.experimental.pallas import tpu_sc as plsc

B = 8
N = 5000
NP = 5120
E = 160000
EP = 163840
F = 128
K = 3
C = 4
CH = 128
NT = 16
RPT = NP // NT
RB = 64
EPT = EP // NT
NCH = EPT // CH
GPC = B // 2


def _sc_spmm(xflat, srcflat, dstflat, wflat):
    mesh = plsc.VectorSubcoreMesh(core_axis_name="c", subcore_axis_name="s")

    @functools.partial(
        pl.kernel,
        mesh=mesh,
        out_type=jax.ShapeDtypeStruct((B * K * NP, F), jnp.float32),
        scratch_types=[
            pltpu.VMEM((NCH, CH), jnp.int32),
            pltpu.VMEM((NCH, CH), jnp.int32),
            pltpu.VMEM((NCH, CH), jnp.float32),
            pltpu.VMEM((CH, F), jnp.float32),
            pltpu.VMEM((CH, F), jnp.float32),
            pltpu.VMEM((RB, F), jnp.float32),
            pltpu.VMEM((RB, F), jnp.float32),
            pltpu.VMEM_SHARED((NP, F), jnp.float32),
            pltpu.SemaphoreType.DMA,
            pltpu.SemaphoreType.DMA,
            pltpu.SemaphoreType.DMA,
            pltpu.SemaphoreType.DMA,
        ],
    )
    def k(x_hbm, src_hbm, dst_hbm, w_hbm, out_hbm,
          src_v, dst_v, w_v, rows_a, rows_b, fin_v, prev_v, acc,
          gs_a, gs_b, ss_a, ss_b):
        cid = lax.axis_index("c")
        sid = lax.axis_index("s")
        stripe = sid * RPT

        def zero_fin(r, carry):
            for f in range(F // 16):
                fin_v[r, pl.ds(f * 16, 16)] = jnp.zeros((16,), jnp.float32)
            return carry

        lax.fori_loop(0, RB, zero_fin, 0)

        def zero_acc(rb, carry):
            pltpu.sync_copy(fin_v, acc.at[pl.ds(stripe + rb * RB, RB)])
            return carry
        lax.fori_loop(0, RPT // RB, zero_acc, 0)
        plsc.subcore_barrier()

        def edge_pass(src_tab, delta):
            def adj(j, c2):
                for i in range(CH // 16):
                    sl = pl.ds(i * 16, 16)
                    src_v[j, sl] = src_v[j, sl] + delta
                return c2
            lax.fori_loop(0, NCH, adj, 0)

            def g_start(j, rows, sem):
                pltpu.async_copy(src_tab.at[src_v.at[j]], rows, sem)

            def g_wait(j, rows, sem):
                pltpu.make_async_copy(src_tab.at[src_v.at[j]], rows, sem).wait()

            def s_start(j, rows, sem):
                pltpu.async_copy(rows, acc.at[dst_v.at[j]], sem, add=True)

            def s_wait(j, rows, sem):
                pltpu.make_async_copy(rows, acc.at[dst_v.at[j]], sem).wait()

            def scale(j, rows):
                def sc16(i, c2):
                    wvec = w_v[j, pl.ds(i * 16, 16)]
                    for jj in range(16):
                        wv = wvec[jj]
                        e = i * 16 + jj
                        for f in range(F // 16):
                            sl = pl.ds(f * 16, 16)
                            rows[e, sl] = rows[e, sl] * wv
                    return c2
                lax.fori_loop(0, CH // 16, sc16, 0)

            g_start(0, rows_a, gs_a)

            def pipe(t, c2):
                a = 2 * t
                bb = 2 * t + 1
                g_wait(a, rows_a, gs_a)

                @pl.when(t > 0)
                def _():
                    s_wait(bb - 2, rows_b, ss_b)
                g_start(bb, rows_b, gs_b)
                scale(a, rows_a)
                s_start(a, rows_a, ss_a)
                g_wait(bb, rows_b, gs_b)
                s_wait(a, rows_a, ss_a)

                @pl.when(t < NCH // 2 - 1)
                def _():
                    g_start(a + 2, rows_a, gs_a)
                scale(bb, rows_b)
                s_start(bb, rows_b, ss_b)
                return c2
            lax.fori_loop(0, NCH // 2, pipe, 0)
            s_wait(NCH - 1, rows_b, ss_b)

        def graph_body(g, carry):
            b = cid * GPC + g
            ebase = b * EP + sid * EPT

            crow = pl.multiple_of(ebase // CH, 8)
            pltpu.sync_copy(src_hbm.at[pl.ds(crow, NCH)], src_v)
            pltpu.sync_copy(dst_hbm.at[pl.ds(crow, NCH)], dst_v)
            pltpu.sync_copy(w_hbm.at[pl.ds(crow, NCH)], w_v)

            for hop in range(K):
                if hop == 0:
                    edge_pass(x_hbm, b * NP)
                elif hop == 1:
                    edge_pass(out_hbm, b * (K - 1) * NP)
                else:
                    edge_pass(out_hbm, NP)
                plsc.subcore_barrier()

                out_row0 = pl.multiple_of((b * K + hop) * NP + stripe, 8)
                if hop == 0:
                    def fin0(rb, c2):
                        r0 = rb * RB
                        pltpu.sync_copy(acc.at[pl.ds(stripe + r0, RB)], fin_v)
                        pltpu.sync_copy(fin_v, out_hbm.at[pl.ds(out_row0 + r0, RB)])
                        lax.fori_loop(0, RB, zero_fin, 0)
                        pltpu.sync_copy(fin_v, acc.at[pl.ds(stripe + r0, RB)])
                        return c2
                    lax.fori_loop(0, RPT // RB, fin0, 0)
                else:
                    if hop == 1:
                        prev_tab = x_hbm
                        prev_row0 = pl.multiple_of(b * NP + stripe, 8)
                    else:
                        prev_tab = out_hbm
                        prev_row0 = pl.multiple_of(
                            (b * K + hop - 2) * NP + stripe, 8)

                    def fin1(rb, c2):
                        r0 = rb * RB
                        pltpu.sync_copy(acc.at[pl.ds(stripe + r0, RB)], fin_v)
                        pltpu.sync_copy(prev_tab.at[pl.ds(prev_row0 + r0, RB)], prev_v)

                        def finrow(r, c3):
                            for f in range(F // 16):
                                sl = pl.ds(f * 16, 16)
                                prev_v[r, sl] = 2.0 * fin_v[r, sl] - prev_v[r, sl]
                                fin_v[r, sl] = jnp.zeros((16,), jnp.float32)
                            return c3
                        lax.fori_loop(0, RB, finrow, 0)
                        pltpu.sync_copy(prev_v, out_hbm.at[pl.ds(out_row0 + r0, RB)])
                        pltpu.sync_copy(fin_v, acc.at[pl.ds(stripe + r0, RB)])
                        return c2
                    lax.fori_loop(0, RPT // RB, fin1, 0)
                plsc.subcore_barrier()
            return carry

        lax.fori_loop(0, GPC, graph_body, 0)

    return k(xflat, srcflat, dstflat, wflat)


def _combine(x, hops, weight, bias):
    R = 1000

    def body(x_ref, h_ref, w_ref, b_ref, o_ref):
        xb = x_ref[0]
        for c in range(C):
            acc = xb * w_ref[c, 0, :][None, :]
            for kk in range(1, K + 1):
                acc = acc + h_ref[0, kk - 1] * w_ref[c, kk, :][None, :]
            o_ref[0, :, c * F:(c + 1) * F] = jnp.maximum(
                acc + b_ref[0, 0, c * F:(c + 1) * F][None, :], 0.0)

    return pl.pallas_call(
        body,
        grid=(B, N // R),
        in_specs=[
            pl.BlockSpec((1, R, F), lambda b, n: (b, n, 0)),
            pl.BlockSpec((1, K, R, F), lambda b, n: (b, 0, n, 0)),
            pl.BlockSpec((C, K + 1, F), lambda b, n: (0, 0, 0)),
            pl.BlockSpec((1, 1, C * F), lambda b, n: (0, 0, 0)),
        ],
        out_specs=pl.BlockSpec((1, R, C * F), lambda b, n: (b, n, 0)),
        out_shape=jax.ShapeDtypeStruct((B, N, C * F), jnp.float32),
    )(x, hops, weight, bias)


def kernel(x, edge_src, edge_dst, edge_w, weight, bias):
    x = x.astype(jnp.float32)
    xflat = jnp.pad(x, ((0, 0), (0, NP - N), (0, 0))).reshape(B * NP, F)
    srcflat = jnp.pad(edge_src.astype(jnp.int32), ((0, 0), (0, EP - E))).reshape(-1, CH)
    dstflat = jnp.pad(edge_dst.astype(jnp.int32), ((0, 0), (0, EP - E))).reshape(-1, CH)
    wflat = jnp.pad(edge_w, ((0, 0), (0, EP - E))).reshape(-1, CH)
    hops = _sc_spmm(xflat, srcflat, dstflat, wflat).reshape(B, K, NP, F)
    return _combine(x, hops, weight, bias)


# speedup vs baseline: 2.9381x; 1.0047x over previous
import functools

import jax
import jax.numpy as jnp
from jax import lax
from jax.experimental import pallas as pl
from jax.experimental.pallas import tpu as pltpu
from jax.experimental.pallas import tpu_sc as plsc

B = 8
N = 5000
NP = 5120
E = 160000
EP = 163840
F = 128
K = 3
C = 4
CH = 128
NT = 16
RPT = NP // NT
RB = 32
EPT = EP // NT
NCH = EPT // CH
GPC = B // 2


def _sc_spmm(xflat, srcflat, dstflat, wflat):
    mesh = plsc.VectorSubcoreMesh(core_axis_name="c", subcore_axis_name="s")

    @functools.partial(
        pl.kernel,
        mesh=mesh,
        out_type=jax.ShapeDtypeStruct((B * K * NP, F), jnp.float32),
        scratch_types=[
            pltpu.VMEM((NCH, CH), jnp.int32),
            pltpu.VMEM((NCH, CH), jnp.int32),
            pltpu.VMEM((NCH, CH), jnp.float32),
            pltpu.VMEM((CH, F), jnp.float32),
            pltpu.VMEM((CH, F), jnp.float32),
            pltpu.VMEM((RB, F), jnp.float32),
            pltpu.VMEM((RB, F), jnp.float32),
            pltpu.VMEM((RB, F), jnp.float32),
            pltpu.VMEM((RB, F), jnp.float32),
            pltpu.VMEM((RB, F), jnp.float32),
            pltpu.VMEM_SHARED((NP, F), jnp.float32),
            pltpu.SemaphoreType.DMA,
            pltpu.SemaphoreType.DMA,
            pltpu.SemaphoreType.DMA,
            pltpu.SemaphoreType.DMA,
            pltpu.SemaphoreType.DMA,
            pltpu.SemaphoreType.DMA,
            pltpu.SemaphoreType.DMA,
            pltpu.SemaphoreType.DMA,
        ],
    )
    def k(x_hbm, src_hbm, dst_hbm, w_hbm, out_hbm,
          src_v, dst_v, w_v, rows_a, rows_b, fin0_v, fin1_v, prev0_v,
          prev1_v, zbuf, acc, gs_a, gs_b, ss_a, ss_b, os_0, os_1, zs_0, zs_1):
        cid = lax.axis_index("c")
        sid = lax.axis_index("s")
        stripe = sid * RPT

        def zero_rows(r, carry):
            for f in range(F // 16):
                zbuf[r, pl.ds(f * 16, 16)] = jnp.zeros((16,), jnp.float32)
            return carry
        lax.fori_loop(0, RB, zero_rows, 0)

        def zero_acc(rb, carry):
            pltpu.sync_copy(zbuf, acc.at[pl.ds(stripe + rb * RB, RB)])
            return carry
        lax.fori_loop(0, RPT // RB, zero_acc, 0)
        plsc.subcore_barrier()

        def edge_pass(src_tab, delta):
            def adj(j, c2):
                for i in range(CH // 16):
                    sl = pl.ds(i * 16, 16)
                    src_v[j, sl] = src_v[j, sl] + delta
                return c2
            lax.fori_loop(0, NCH, adj, 0)

            def g_start(j, rows, sem):
                pltpu.async_copy(src_tab.at[src_v.at[j]], rows, sem)

            def g_wait(j, rows, sem):
                pltpu.make_async_copy(src_tab.at[src_v.at[j]], rows, sem).wait()

            def s_start(j, rows, sem):
                pltpu.async_copy(rows, acc.at[dst_v.at[j]], sem, add=True)

            def s_wait(j, rows, sem):
                pltpu.make_async_copy(rows, acc.at[dst_v.at[j]], sem).wait()

            def scale(j, rows):
                def sc16(i, c2):
                    wvec = w_v[j, pl.ds(i * 16, 16)]
                    for jj in range(16):
                        wv = wvec[jj]
                        e = i * 16 + jj
                        for f in range(F // 16):
                            sl = pl.ds(f * 16, 16)
                            rows[e, sl] = rows[e, sl] * wv
                    return c2
                lax.fori_loop(0, CH // 16, sc16, 0)

            g_start(0, rows_a, gs_a)

            def pipe(t, c2):
                a = 2 * t
                bb = 2 * t + 1
                g_wait(a, rows_a, gs_a)

                @pl.when(t > 0)
                def _():
                    s_wait(bb - 2, rows_b, ss_b)
                g_start(bb, rows_b, gs_b)
                scale(a, rows_a)
                s_start(a, rows_a, ss_a)
                g_wait(bb, rows_b, gs_b)
                s_wait(a, rows_a, ss_a)

                @pl.when(t < NCH // 2 - 1)
                def _():
                    g_start(a + 2, rows_a, gs_a)
                scale(bb, rows_b)
                s_start(bb, rows_b, ss_b)
                return c2
            lax.fori_loop(0, NCH // 2, pipe, 0)
            s_wait(NCH - 1, rows_b, ss_b)

        def graph_body(g, carry):
            b = cid * GPC + g
            ebase = b * EP + sid * EPT

            crow = pl.multiple_of(ebase // CH, 8)
            pltpu.sync_copy(src_hbm.at[pl.ds(crow, NCH)], src_v)
            pltpu.sync_copy(dst_hbm.at[pl.ds(crow, NCH)], dst_v)
            pltpu.sync_copy(w_hbm.at[pl.ds(crow, NCH)], w_v)

            for hop in range(K):
                if hop == 0:
                    edge_pass(x_hbm, b * NP)
                elif hop == 1:
                    edge_pass(out_hbm, b * (K - 1) * NP)
                else:
                    edge_pass(out_hbm, NP)
                plsc.subcore_barrier()

                out_row0 = pl.multiple_of((b * K + hop) * NP + stripe, 8)
                if hop == 0:
                    prev_tab = x_hbm
                    prev_row0 = 0
                elif hop == 1:
                    prev_tab = x_hbm
                    prev_row0 = pl.multiple_of(b * NP + stripe, 8)
                else:
                    prev_tab = out_hbm
                    prev_row0 = pl.multiple_of(
                        (b * K + hop - 2) * NP + stripe, 8)

                sets = ((fin0_v, prev0_v, gs_a, ss_a, os_0, zs_0),
                        (fin1_v, prev1_v, gs_b, ss_b, os_1, zs_1))
                nsb = RPT // RB

                def in_start(rb, s):
                    pltpu.async_copy(acc.at[pl.ds(stripe + rb * RB, RB)],
                                     s[0], s[2])
                    if hop > 0:
                        pltpu.async_copy(
                            prev_tab.at[pl.ds(prev_row0 + rb * RB, RB)],
                            s[1], s[3])

                def in_wait(rb, s):
                    pltpu.make_async_copy(
                        acc.at[pl.ds(stripe + rb * RB, RB)], s[0], s[2]).wait()
                    if hop > 0:
                        pltpu.make_async_copy(
                            prev_tab.at[pl.ds(prev_row0 + rb * RB, RB)],
                            s[1], s[3]).wait()

                def out_src(s):
                    return s[1] if hop > 0 else s[0]

                def out_wait(rb, s):
                    pltpu.make_async_copy(
                        out_src(s), out_hbm.at[pl.ds(out_row0 + rb * RB, RB)],
                        s[4]).wait()
                    pltpu.make_async_copy(
                        zbuf, acc.at[pl.ds(stripe + rb * RB, RB)], s[5]).wait()

                in_start(0, sets[0])

                def fin_u(u, c2):
                    for par in range(2):
                        s = sets[par]
                        rb = 2 * u + par
                        in_wait(rb, s)
                        if hop > 0:
                            fin_b, prev_b = s[0], s[1]

                            def finrow(r, c3):
                                for f in range(F // 16):
                                    sl = pl.ds(f * 16, 16)
                                    prev_b[r, sl] = (2.0 * fin_b[r, sl]
                                                     - prev_b[r, sl])
                                return c3
                            lax.fori_loop(0, RB, finrow, 0)
                        pltpu.async_copy(
                            out_src(s),
                            out_hbm.at[pl.ds(out_row0 + rb * RB, RB)], s[4])
                        pltpu.async_copy(
                            zbuf, acc.at[pl.ds(stripe + rb * RB, RB)], s[5])
                        s2 = sets[1 - par]

                        @pl.when(rb + 1 < nsb)
                        def _():
                            @pl.when(rb >= 1)
                            def _():
                                out_wait(rb - 1, s2)
                            in_start(rb + 1, s2)
                    return c2
                lax.fori_loop(0, nsb // 2, fin_u, 0)
                out_wait(nsb - 2, sets[0])
                out_wait(nsb - 1, sets[1])
                plsc.subcore_barrier()
            return carry

        lax.fori_loop(0, GPC, graph_body, 0)

    return k(xflat, srcflat, dstflat, wflat)


def _combine(x, hops, weight, bias):
    R = 1000

    def body(x_ref, h_ref, w_ref, b_ref, o_ref):
        xb = x_ref[0]
        for c in range(C):
            acc = xb * w_ref[c, 0, :][None, :]
            for kk in range(1, K + 1):
                acc = acc + h_ref[0, kk - 1] * w_ref[c, kk, :][None, :]
            o_ref[0, :, c * F:(c + 1) * F] = jnp.maximum(
                acc + b_ref[0, 0, c * F:(c + 1) * F][None, :], 0.0)

    return pl.pallas_call(
        body,
        grid=(B, N // R),
        in_specs=[
            pl.BlockSpec((1, R, F), lambda b, n: (b, n, 0)),
            pl.BlockSpec((1, K, R, F), lambda b, n: (b, 0, n, 0)),
            pl.BlockSpec((C, K + 1, F), lambda b, n: (0, 0, 0)),
            pl.BlockSpec((1, 1, C * F), lambda b, n: (0, 0, 0)),
        ],
        out_specs=pl.BlockSpec((1, R, C * F), lambda b, n: (b, n, 0)),
        out_shape=jax.ShapeDtypeStruct((B, N, C * F), jnp.float32),
    )(x, hops, weight, bias)


def kernel(x, edge_src, edge_dst, edge_w, weight, bias):
    x = x.astype(jnp.float32)
    xflat = jnp.pad(x, ((0, 0), (0, NP - N), (0, 0))).reshape(B * NP, F)
    srcflat = jnp.pad(edge_src.astype(jnp.int32), ((0, 0), (0, EP - E))).reshape(-1, CH)
    dstflat = jnp.pad(edge_dst.astype(jnp.int32), ((0, 0), (0, EP - E))).reshape(-1, CH)
    wflat = jnp.pad(edge_w, ((0, 0), (0, EP - E))).reshape(-1, CH)
    hops = _sc_spmm(xflat, srcflat, dstflat, wflat).reshape(B, K, NP, F)
    return _combine(x, hops, weight, bias)


# speedup vs baseline: 2.9475x; 1.0032x over previous
import functools

import jax
import jax.numpy as jnp
from jax import lax
from jax.experimental import pallas as pl
from jax.experimental.pallas import tpu as pltpu
from jax.experimental.pallas import tpu_sc as plsc

B = 8
N = 5000
NP = 5120
E = 160000
EP = 163840
F = 128
K = 3
C = 4
CH = 128
NT = 16
RPT = NP // NT
RB = 32
EPT = EP // NT
NCH = EPT // CH
GPC = B // 2


def _sc_spmm(xflat, srcflat, dstflat, wflat):
    mesh = plsc.VectorSubcoreMesh(core_axis_name="c", subcore_axis_name="s")

    @functools.partial(
        pl.kernel,
        mesh=mesh,
        out_type=jax.ShapeDtypeStruct((B * K * NP, F), jnp.float32),
        scratch_types=[
            pltpu.VMEM((4, CH), jnp.int32),
            pltpu.VMEM((4, CH), jnp.int32),
            pltpu.VMEM((4, CH), jnp.float32),
            pltpu.VMEM((CH, F), jnp.float32),
            pltpu.VMEM((CH, F), jnp.float32),
            pltpu.VMEM((CH, F), jnp.float32),
            pltpu.VMEM((CH, F), jnp.float32),
            pltpu.VMEM((RB, F), jnp.float32),
            pltpu.VMEM((RB, F), jnp.float32),
            pltpu.VMEM((RB, F), jnp.float32),
            pltpu.VMEM((RB, F), jnp.float32),
            pltpu.VMEM((RB, F), jnp.float32),
            pltpu.VMEM_SHARED((NP, F), jnp.float32),
        ] + [pltpu.SemaphoreType.DMA] * 16,
    )
    def k(x_hbm, src_hbm, dst_hbm, w_hbm, out_hbm,
          src_c, dst_c, w_c, rows_0, rows_1, rows_2, rows_3,
          fin0_v, fin1_v, prev0_v, prev1_v, zbuf, acc,
          is_0, is_1, is_2, is_3,
          gs_0, gs_1, gs_2, gs_3, ss_0, ss_1, ss_2, ss_3,
          os_0, os_1, zs_0, zs_1):
        cid = lax.axis_index("c")
        sid = lax.axis_index("s")
        stripe = sid * RPT
        isems = (is_0, is_1, is_2, is_3)
        rbufs = (rows_0, rows_1, rows_2, rows_3)
        gsems = (gs_0, gs_1, gs_2, gs_3)
        ssems = (ss_0, ss_1, ss_2, ss_3)

        def zero_rows(r, carry):
            for f in range(F // 16):
                zbuf[r, pl.ds(f * 16, 16)] = jnp.zeros((16,), jnp.float32)
            return carry
        lax.fori_loop(0, RB, zero_rows, 0)

        def zero_acc(rb, carry):
            pltpu.sync_copy(zbuf, acc.at[pl.ds(stripe + rb * RB, RB)])
            return carry
        lax.fori_loop(0, RPT // RB, zero_acc, 0)
        plsc.subcore_barrier()

        def edge_pass(src_tab, delta, crow):
            def idx_start(row, k8):
                pltpu.async_copy(src_hbm.at[row], src_c.at[k8], isems[k8])
                pltpu.async_copy(dst_hbm.at[row], dst_c.at[k8], isems[k8])
                pltpu.async_copy(w_hbm.at[row], w_c.at[k8], isems[k8])

            def idx_wait(row, k8):
                pltpu.make_async_copy(src_hbm.at[row], src_c.at[k8],
                                      isems[k8]).wait()
                pltpu.make_async_copy(dst_hbm.at[row], dst_c.at[k8],
                                      isems[k8]).wait()
                pltpu.make_async_copy(w_hbm.at[row], w_c.at[k8],
                                      isems[k8]).wait()
                for i in range(CH // 16):
                    sl = pl.ds(i * 16, 16)
                    src_c[k8, sl] = src_c[k8, sl] + delta

            def g_start(p, k8):
                pltpu.async_copy(src_tab.at[src_c.at[k8]], rbufs[p], gsems[p])

            def g_wait(p, k8):
                pltpu.make_async_copy(src_tab.at[src_c.at[k8]], rbufs[p],
                                      gsems[p]).wait()

            def s_start(p, k8):
                pltpu.async_copy(rbufs[p], acc.at[dst_c.at[k8]], ssems[p],
                                 add=True)

            def s_wait(p, k8):
                pltpu.make_async_copy(rbufs[p], acc.at[dst_c.at[k8]],
                                      ssems[p]).wait()

            def scale(p, k8):
                rows = rbufs[p]

                def sc16(i, c2):
                    wvec = w_c[k8, pl.ds(i * 16, 16)]
                    for jj in range(16):
                        wv = wvec[jj]
                        e = i * 16 + jj
                        for f in range(F // 16):
                            sl = pl.ds(f * 16, 16)
                            rows[e, sl] = rows[e, sl] * wv
                    return c2
                lax.fori_loop(0, CH // 16, sc16, 0)

            idx_start(crow, 0)
            idx_start(crow + 1, 1)
            idx_wait(crow, 0)
            g_start(0, 0)

            def pipe(t, c2):
                for jj in range(4):
                    j = 4 * t + jj

                    g_wait(jj, jj)

                    @pl.when(j + 1 < NCH)
                    def _():
                        idx_wait(crow + j + 1, (jj + 1) % 4)

                        @pl.when(j >= 2)
                        def _():
                            s_wait((jj + 2) % 4, (jj + 2) % 4)
                        g_start((jj + 1) % 4, (jj + 1) % 4)

                    @pl.when(j + 2 < NCH)
                    def _():
                        idx_start(crow + j + 2, (jj + 2) % 4)
                    scale(jj, jj)
                    s_start(jj, jj)
                return c2
            lax.fori_loop(0, NCH // 4, pipe, 0)
            for jt in range(NCH - 3, NCH):
                s_wait(jt % 4, jt % 4)

        def graph_body(g, carry):
            b = cid * GPC + g
            ebase = b * EP + sid * EPT
            crow = pl.multiple_of(ebase // CH, 8)

            for hop in range(K):
                if hop == 0:
                    edge_pass(x_hbm, b * NP, crow)
                elif hop == 1:
                    edge_pass(out_hbm, b * K * NP, crow)
                else:
                    edge_pass(out_hbm, (b * K + 1) * NP, crow)
                plsc.subcore_barrier()

                out_row0 = pl.multiple_of((b * K + hop) * NP + stripe, 8)
                if hop == 0:
                    prev_tab = x_hbm
                    prev_row0 = 0
                elif hop == 1:
                    prev_tab = x_hbm
                    prev_row0 = pl.multiple_of(b * NP + stripe, 8)
                else:
                    prev_tab = out_hbm
                    prev_row0 = pl.multiple_of(
                        (b * K + hop - 2) * NP + stripe, 8)

                sets = ((fin0_v, prev0_v, gs_0, ss_0, os_0, zs_0),
                        (fin1_v, prev1_v, gs_1, ss_1, os_1, zs_1))
                nsb = RPT // RB

                def in_start(rb, s):
                    pltpu.async_copy(acc.at[pl.ds(stripe + rb * RB, RB)],
                                     s[0], s[2])
                    if hop > 0:
                        pltpu.async_copy(
                            prev_tab.at[pl.ds(prev_row0 + rb * RB, RB)],
                            s[1], s[3])

                def in_wait(rb, s):
                    pltpu.make_async_copy(
                        acc.at[pl.ds(stripe + rb * RB, RB)], s[0], s[2]).wait()
                    if hop > 0:
                        pltpu.make_async_copy(
                            prev_tab.at[pl.ds(prev_row0 + rb * RB, RB)],
                            s[1], s[3]).wait()

                def out_src(s):
                    return s[1] if hop > 0 else s[0]

                def out_wait(rb, s):
                    pltpu.make_async_copy(
                        out_src(s), out_hbm.at[pl.ds(out_row0 + rb * RB, RB)],
                        s[4]).wait()
                    pltpu.make_async_copy(
                        zbuf, acc.at[pl.ds(stripe + rb * RB, RB)], s[5]).wait()

                in_start(0, sets[0])

                def fin_u(u, c2):
                    for par in range(2):
                        s = sets[par]
                        rb = 2 * u + par
                        in_wait(rb, s)
                        if hop > 0:
                            fin_b, prev_b = s[0], s[1]

                            def finrow(r, c3):
                                for f in range(F // 16):
                                    sl = pl.ds(f * 16, 16)
                                    prev_b[r, sl] = (2.0 * fin_b[r, sl]
                                                     - prev_b[r, sl])
                                return c3
                            lax.fori_loop(0, RB, finrow, 0)
                        pltpu.async_copy(
                            out_src(s),
                            out_hbm.at[pl.ds(out_row0 + rb * RB, RB)], s[4])
                        pltpu.async_copy(
                            zbuf, acc.at[pl.ds(stripe + rb * RB, RB)], s[5])
                        s2 = sets[1 - par]

                        @pl.when(rb + 1 < nsb)
                        def _():
                            @pl.when(rb >= 1)
                            def _():
                                out_wait(rb - 1, s2)
                            in_start(rb + 1, s2)
                    return c2
                lax.fori_loop(0, nsb // 2, fin_u, 0)
                out_wait(nsb - 2, sets[0])
                out_wait(nsb - 1, sets[1])
                plsc.subcore_barrier()
            return carry

        lax.fori_loop(0, GPC, graph_body, 0)

    return k(xflat, srcflat, dstflat, wflat)


def _combine(x, hops, weight, bias):
    R = 1000

    def body(x_ref, h_ref, w_ref, b_ref, o_ref):
        xb = x_ref[0]
        for c in range(C):
            acc = xb * w_ref[c, 0, :][None, :]
            for kk in range(1, K + 1):
                acc = acc + h_ref[0, kk - 1] * w_ref[c, kk, :][None, :]
            o_ref[0, :, c * F:(c + 1) * F] = jnp.maximum(
                acc + b_ref[0, 0, c * F:(c + 1) * F][None, :], 0.0)

    return pl.pallas_call(
        body,
        grid=(B, N // R),
        in_specs=[
            pl.BlockSpec((1, R, F), lambda b, n: (b, n, 0)),
            pl.BlockSpec((1, K, R, F), lambda b, n: (b, 0, n, 0)),
            pl.BlockSpec((C, K + 1, F), lambda b, n: (0, 0, 0)),
            pl.BlockSpec((1, 1, C * F), lambda b, n: (0, 0, 0)),
        ],
        out_specs=pl.BlockSpec((1, R, C * F), lambda b, n: (b, n, 0)),
        out_shape=jax.ShapeDtypeStruct((B, N, C * F), jnp.float32),
    )(x, hops, weight, bias)


def kernel(x, edge_src, edge_dst, edge_w, weight, bias):
    x = x.astype(jnp.float32)
    xflat = jnp.pad(x, ((0, 0), (0, NP - N), (0, 0))).reshape(B * NP, F)
    srcflat = jnp.pad(edge_src.astype(jnp.int32), ((0, 0), (0, EP - E))).reshape(-1, CH)
    dstflat = jnp.pad(edge_dst.astype(jnp.int32), ((0, 0), (0, EP - E))).reshape(-1, CH)
    wflat = jnp.pad(edge_w, ((0, 0), (0, EP - E))).reshape(-1, CH)
    hops = _sc_spmm(xflat, srcflat, dstflat, wflat).reshape(B, K, NP, F)
    return _combine(x, hops, weight, bias)
